# baseline (speedup 1.0000x reference)
import functools

import jax
import jax.numpy as jnp
from jax import lax
from jax.experimental import pallas as pl
from jax.experimental.pallas import tpu as pltpu
from jax.experimental.pallas import tpu_sc as plsc

N_CONCEPTS = 1000000
NLAYERS = 2
HIDDEN = 64
BATCH = 16384

_info = plsc.get_sparse_core_info()
NC = _info.num_cores
NS = _info.num_subcores
NW = NC * NS
B_PER_W = BATCH // NW
CHUNK = 128
NPAIR = 2

NBUF = 5
NGATHER = 2 * 2 * NPAIR


def _sc_body(x_ref, c_ref, h_ref, c_out, h_out, idx_v, bufs, pads, gsems,
             wsems):
    wid = lax.axis_index("s") * NC + lax.axis_index("c")
    row0 = pl.multiple_of((wid // 2) * 8, 8)
    pltpu.sync_copy(x_ref.at[pl.ds(row0, 8), :], idx_v)
    lbase = (wid % 2) * 4

    tables = (c_ref, h_ref)
    outs = (c_out, h_out)

    def start_gather(g):
        t, c, p = g // 4, (g // 2) % 2, g % 2
        b = g % NBUF
        return pltpu.async_copy(
            tables[t].at[idx_v.at[lbase + 2 * p + c]],
            bufs.at[b],
            gsems.at[b],
        )

    gh = [start_gather(g) for g in range(NBUF)]
    wh = {}

    for pair in range(2 * NPAIR):
        t, c = pair // NPAIR, pair % NPAIR
        be, bo = (2 * pair) % NBUF, (2 * pair + 1) % NBUF
        gh[2 * pair].wait()
        gh[2 * pair + 1].wait()
        s = 0
        if pair >= 1:
            wh[(s, 0)].wait()
            wh[(s, 1)].wait()

        @plsc.parallel_loop(0, CHUNK, unroll=8)
        def rep(r, _be=be, _bo=bo, _s=s):
            for l in range(NLAYERS):
                for k in range(HIDDEN // 16):
                    pads[_s, l, r, pl.ds(16 * k, 16)] = bufs[
                        _be, r, pl.ds(l * HIDDEN + 16 * k, 16)
                    ]
                    pads[_s, l, r, pl.ds(HIDDEN + 16 * k, 16)] = bufs[
                        _bo, r, pl.ds(l * HIDDEN + 16 * k, 16)
                    ]

        base = wid * (NPAIR * CHUNK) + c * CHUNK
        for l in range(NLAYERS):
            wh[(s, l)] = pltpu.async_copy(
                pads.at[s, l],
                outs[t].at[l, pl.ds(base, CHUNK), :],
                wsems.at[s * NLAYERS + l],
            )
        ng = 2 * pair + NBUF
        for g in (ng, ng + 1):
            if g < NGATHER:
                gh.append(start_gather(g))
    wh[(0, 0)].wait()
    wh[(0, 1)].wait()


@functools.partial(jax.jit, static_argnums=())
def kernel(x, c_table, h_table):
    x_st = (
        x.reshape(NW, NPAIR, CHUNK, 2).transpose(0, 3, 1, 2).reshape(128, 128)
    )
    out_sds = jax.ShapeDtypeStruct(
        (NLAYERS, BATCH // 2, 2 * HIDDEN), jnp.float32
    )
    run = pl.kernel(
        _sc_body,
        out_type=(out_sds, out_sds),
        mesh=plsc.VectorSubcoreMesh(core_axis_name="c", subcore_axis_name="s"),
        scratch_types=[
            pltpu.VMEM((8, CHUNK), jnp.int32),
            pltpu.VMEM((NBUF, CHUNK, NLAYERS * HIDDEN), jnp.float32),
            pltpu.VMEM((1, NLAYERS, CHUNK, 2 * HIDDEN), jnp.float32),
            pltpu.SemaphoreType.DMA((NBUF,)),
            pltpu.SemaphoreType.DMA((NLAYERS,)),
        ],
    )
    c_pack, h_pack = run(x_st, c_table, h_table)
    c_init = c_pack.reshape(NLAYERS, BATCH, HIDDEN)
    h_init = h_pack.reshape(NLAYERS, BATCH, HIDDEN)
    return (c_init, h_init)


# speedup vs baseline: 1.5685x; 1.5685x over previous
import functools

import jax
import jax.numpy as jnp
from jax import lax
from jax.experimental import pallas as pl
from jax.experimental.pallas import tpu as pltpu
from jax.experimental.pallas import tpu_sc as plsc

N_CONCEPTS = 1000000
NLAYERS = 2
HIDDEN = 64
BATCH = 16384

_info = plsc.get_sparse_core_info()
NC = _info.num_cores
NS = _info.num_subcores
NW = NC * NS
B_PER_W = BATCH // NW
CHUNK = 128
NCHUNK = B_PER_W // CHUNK


NBUF = 6
NGATHER = 2 * NCHUNK


def _sc_body(x_ref, c_ref, h_ref, c_out, h_out, idx_v, bufs, gsems, wsems):
    wid = lax.axis_index("s") * NC + lax.axis_index("c")
    row0 = pl.multiple_of((wid // 2) * (2 * NCHUNK), 8)
    pltpu.sync_copy(x_ref.at[pl.ds(row0, 2 * NCHUNK), :], idx_v)

    tables = (c_ref, h_ref)
    outs = (c_out, h_out)

    def start_gather(g):
        t, j = divmod(g, NCHUNK)
        b = g % NBUF
        return pltpu.async_copy(
            tables[t].at[idx_v.at[(wid % 2) * NCHUNK + j]],
            bufs.at[b],
            gsems.at[b],
        )

    gh = [start_gather(g) for g in range(NBUF)]
    gh += [None] * (NGATHER - NBUF)
    wh = [None] * NBUF

    for g in range(NGATHER):
        t, j = divmod(g, NCHUNK)
        b = g % NBUF
        gh[g].wait()
        base = wid * B_PER_W + j * CHUNK
        wh[b] = pltpu.async_copy(
            bufs.at[b], outs[t].at[pl.ds(base, CHUNK), :], wsems.at[b]
        )
        ng = g + 3
        if NBUF <= ng < NGATHER:
            wh[ng % NBUF].wait()
            gh[ng] = start_gather(ng)
    for g in range(NGATHER - NBUF, NGATHER):
        wh[g % NBUF].wait()


@functools.partial(jax.jit, static_argnums=())
def kernel(x, c_table, h_table):
    x3 = x.reshape(BATCH // CHUNK, CHUNK)
    out_sds = jax.ShapeDtypeStruct((BATCH, NLAYERS * HIDDEN), jnp.float32)
    run = pl.kernel(
        _sc_body,
        out_type=(out_sds, out_sds),
        mesh=plsc.VectorSubcoreMesh(core_axis_name="c", subcore_axis_name="s"),
        scratch_types=[
            pltpu.VMEM((2 * NCHUNK, CHUNK), jnp.int32),
            pltpu.VMEM((NBUF, CHUNK, NLAYERS * HIDDEN), jnp.float32),
            pltpu.SemaphoreType.DMA((NBUF,)),
            pltpu.SemaphoreType.DMA((NBUF,)),
        ],
    )
    c_rows, h_rows = run(x3, c_table, h_table)
    c_init = jnp.swapaxes(c_rows.reshape(BATCH, NLAYERS, HIDDEN), 0, 1)
    h_init = jnp.swapaxes(h_rows.reshape(BATCH, NLAYERS, HIDDEN), 0, 1)
    return (c_init, h_init)
